# Initial kernel scaffold; baseline (speedup 1.0000x reference)
import jax
import jax.numpy as jnp
from jax.experimental import pallas as pl


def kernel(depth_pred, depth_gt, depth_mask, bin_edges):
    raise NotImplementedError("write your pallas kernel here")


# speedup vs baseline: 1.0478x; 1.0478x over previous
import jax
import jax.numpy as jnp
from jax.experimental import pallas as pl
from jax.experimental.pallas import tpu as pltpu

_BIG = 1e10
_CHUNK = 2048


def _chamfer_body(edges_ref, tgt_ref, mask_ref, out_ref):
    n = pl.program_id(0)
    num_n = pl.num_programs(0)
    L = tgt_ref.shape[2]
    P = edges_ref.shape[2] - 1

    centers = 0.5 * (edges_ref[0, 0, 1:] + edges_ref[0, 0, :-1])
    cb = jax.lax.dot_general(
        centers.reshape(1, P), jnp.ones((1, _CHUNK), jnp.float32),
        (((0,), (0,)), ((), ())), preferred_element_type=jnp.float32,
        precision=jax.lax.Precision.HIGHEST,
    )

    minx = jnp.full((P, 1), _BIG, dtype=jnp.float32)
    sumy = jnp.zeros((1, 1), dtype=jnp.float32)
    cnt = jnp.zeros((1, 1), dtype=jnp.float32)
    for c in range(L // _CHUNK):
        tt = tgt_ref[0, 0, pl.ds(c * _CHUNK, _CHUNK)].reshape(1, _CHUNK)
        mm = mask_ref[0, 0, pl.ds(c * _CHUNK, _CHUNK)].reshape(1, _CHUNK)
        d2 = (cb - tt) ** 2
        d2m = jnp.where(mm > 0, d2, _BIG)
        minx = jnp.minimum(minx, jnp.min(d2m, axis=1, keepdims=True))
        miny = jnp.min(d2, axis=0, keepdims=True)
        sumy = sumy + jnp.sum(miny * mm, keepdims=True)
        cnt = cnt + jnp.sum(mm, keepdims=True)

    cham_x = jnp.sum(minx, keepdims=True).reshape(1, 1) / P
    cham_y = sumy / jnp.maximum(cnt, 1.0)

    @pl.when(n == 0)
    def _():
        out_ref[0, 0] = 0.0

    out_ref[0, 0] += (cham_x[0, 0] + cham_y[0, 0]) / num_n


def kernel(depth_pred, depth_gt, depth_mask, bin_edges):
    del depth_pred
    n = bin_edges.shape[0]
    tgt = depth_gt.reshape(n, 1, -1)
    maskf = depth_mask.reshape(n, 1, -1).astype(jnp.float32)
    L = tgt.shape[2]
    pe = bin_edges.shape[1]
    edges3 = bin_edges.reshape(n, 1, pe)

    out = pl.pallas_call(
        _chamfer_body,
        grid=(n,),
        in_specs=[
            pl.BlockSpec((1, 1, pe), lambda i: (i, 0, 0)),
            pl.BlockSpec((1, 1, L), lambda i: (i, 0, 0)),
            pl.BlockSpec((1, 1, L), lambda i: (i, 0, 0)),
        ],
        out_specs=pl.BlockSpec((1, 1), lambda i: (0, 0), memory_space=pltpu.SMEM),
        out_shape=jax.ShapeDtypeStruct((1, 1), jnp.float32),
    )(edges3, tgt, maskf)
    return out[0, 0]
